# baseline (speedup 1.0000x reference)
import functools

import jax
import jax.numpy as jnp
from jax import lax
from jax.experimental import pallas as pl
from jax.experimental.pallas import tpu as pltpu
from jax.experimental.pallas import tpu_sc as plsc

N = 10000
D = 128
DH = D // 2
WH = DH // 2
E = 320000
LN_EPS = 1e-5

K = 112
CHUNKS = 179
EPW = CHUNKS * K
E_PAD = 16 * EPW

_f32 = jnp.float32
_i32 = jnp.int32
_u32 = jnp.uint32
_bf16 = jnp.bfloat16

_sc_mesh = plsc.VectorSubcoreMesh(core_axis_name="c", subcore_axis_name="s")


@functools.partial(
    pl.kernel,
    out_type=jax.ShapeDtypeStruct((2 * N,), _f32),
    mesh=_sc_mesh,
    scratch_types=[
        pltpu.VMEM((CHUNKS, K), _i32),
        pltpu.VMEM((CHUNKS, K), _f32),
        pltpu.VMEM((N,), _f32),
        pltpu.VMEM_SHARED((N,), _f32),
    ],
)
def _deg_sc(col_hbm, w_hbm, dp_hbm, col_v, w_v, dvmem, dacc):
    cid = lax.axis_index("c")
    sid = lax.axis_index("s")

    pltpu.sync_copy(col_hbm.at[sid], col_v)
    pltpu.sync_copy(w_hbm.at[sid], w_v)

    @pl.when(sid == 0)
    def _zero():
        zero16 = jnp.zeros((16,), _f32)

        def zr(i, _):
            dvmem[pl.ds(i * 16, 16)] = zero16
            return 0

        lax.fori_loop(0, N // 16, zr, 0)
        pltpu.sync_copy(dvmem, dacc)

    plsc.subcore_barrier()

    lo = jnp.where(cid == 0, 0, 90)
    hi = jnp.where(cid == 0, 90, CHUNKS)

    def chunk(g, _):
        pltpu.sync_copy(w_v.at[g], dacc.at[col_v.at[g]], add=True)
        return 0

    lax.fori_loop(lo, hi, chunk, 0)

    plsc.subcore_barrier()

    @pl.when(sid == 0)
    def _writeout():
        pltpu.sync_copy(dacc, dvmem)
        pltpu.sync_copy(dvmem, dp_hbm.at[pl.ds(cid * N, N)])


@functools.partial(
    pl.kernel,
    out_type=jax.ShapeDtypeStruct((2, N, DH), _f32),
    mesh=_sc_mesh,
    scratch_types=[
        pltpu.VMEM((CHUNKS, K), _i32),
        pltpu.VMEM((CHUNKS, K), _i32),
        pltpu.VMEM((CHUNKS, K), _f32),
        pltpu.VMEM((K, WH), _u32),
        pltpu.VMEM((K, WH), _u32),
        pltpu.VMEM((K, DH), _f32),
        pltpu.VMEM((K, DH), _f32),
        pltpu.VMEM_SHARED((N, DH), _f32),
        pltpu.SemaphoreType.DMA,
        pltpu.SemaphoreType.DMA,
        pltpu.SemaphoreType.DMA,
        pltpu.SemaphoreType.DMA,
    ],
    compiler_params=pltpu.CompilerParams(use_tc_tiling_on_sc=False,
                                         needs_layout_passes=False),
)
def _msg_sc(hp_hbm, row_hbm, col_hbm, w_hbm, s_hbm,
            row_v, col_v, w_v, g0, g1, s0, s1, acc,
            sem_g0, sem_g1, sem_s0, sem_s1):
    cid = lax.axis_index("c")
    sid = lax.axis_index("s")
    wid = cid * 16 + sid

    pltpu.sync_copy(row_hbm.at[wid], row_v)
    pltpu.sync_copy(col_hbm.at[sid], col_v)
    pltpu.sync_copy(w_hbm.at[sid], w_v)

    zero16 = jnp.zeros((16,), _f32)

    def zr(i, _):
        for q in range(DH // 16):
            s0[i, pl.ds(q * 16, 16)] = zero16
        return 0

    lax.fori_loop(0, K, zr, 0)
    base = sid * 624
    for j in range(5):
        pltpu.sync_copy(s0, acc.at[pl.ds(base + j * K, K)])

    @pl.when(sid == 15)
    def _ztail_full():
        pltpu.sync_copy(s0.at[pl.ds(0, 80)], acc.at[pl.ds(base + 5 * K, 80)])

    @pl.when(sid != 15)
    def _ztail_part():
        pltpu.sync_copy(s0.at[pl.ds(0, 64)], acc.at[pl.ds(base + 5 * K, 64)])

    plsc.subcore_barrier()

    mask_hi = jnp.full((16,), 0xFFFF0000, dtype=_u32)

    def scale(gb, sb, g):
        def grp(j, _):
            w16 = w_v[g, pl.ds(j * 16, 16)]
            for t in range(16):
                e = j * 16 + t
                sw = w16[t]
                for q in range(WH // 16):
                    v = gb[e, pl.ds(q * 16, 16)]
                    fa = plsc.bitcast(v << 16, _f32)
                    fb = plsc.bitcast(v & mask_hi, _f32)
                    sb[e, pl.ds(q * 32, 16)] = fa * sw
                    sb[e, pl.ds(q * 32 + 16, 16)] = fb * sw
            return 0

        lax.fori_loop(0, K // 16, grp, 0)

    pltpu.async_copy(hp_hbm.at[row_v.at[0]], g0, sem_g0)
    pltpu.async_copy(hp_hbm.at[row_v.at[1]], g1, sem_g1)

    def pipe(i, _):
        ga = 2 * i
        gb_ = 2 * i + 1

        pltpu.make_async_copy(hp_hbm.at[row_v.at[ga]], g0, sem_g0).wait()

        @pl.when(i >= 1)
        def _drain_a():
            pltpu.make_async_copy(s0, acc.at[col_v.at[ga - 2]], sem_s0).wait()

        scale(g0, s0, ga)
        pltpu.async_copy(s0, acc.at[col_v.at[ga]], sem_s0, add=True)
        pltpu.async_copy(hp_hbm.at[row_v.at[ga + 2]], g0, sem_g0)

        pltpu.make_async_copy(hp_hbm.at[row_v.at[gb_]], g1, sem_g1).wait()

        @pl.when(i >= 1)
        def _drain_b():
            pltpu.make_async_copy(s1, acc.at[col_v.at[gb_ - 2]], sem_s1).wait()

        scale(g1, s1, gb_)
        pltpu.async_copy(s1, acc.at[col_v.at[gb_]], sem_s1, add=True)

        @pl.when(i < (CHUNKS - 1) // 2 - 1)
        def _next_b():
            pltpu.async_copy(hp_hbm.at[row_v.at[gb_ + 2]], g1, sem_g1)

        return 0

    lax.fori_loop(0, (CHUNKS - 1) // 2, pipe, 0)

    last = CHUNKS - 1
    pltpu.make_async_copy(hp_hbm.at[row_v.at[last]], g0, sem_g0).wait()
    pltpu.make_async_copy(s0, acc.at[col_v.at[last - 2]], sem_s0).wait()
    scale(g0, s0, last)
    pltpu.async_copy(s0, acc.at[col_v.at[last]], sem_s0, add=True)
    pltpu.make_async_copy(s1, acc.at[col_v.at[last - 1]], sem_s1).wait()
    pltpu.make_async_copy(s0, acc.at[col_v.at[last]], sem_s0).wait()

    plsc.subcore_barrier()

    for j in range(5):
        pltpu.sync_copy(acc.at[pl.ds(base + j * K, K)],
                        s_hbm.at[cid, pl.ds(base + j * K, K)])

    @pl.when(sid == 15)
    def _wtail_full():
        pltpu.sync_copy(acc.at[pl.ds(base + 5 * K, 80)],
                        s_hbm.at[cid, pl.ds(base + 5 * K, 80)])

    @pl.when(sid != 15)
    def _wtail_part():
        pltpu.sync_copy(acc.at[pl.ds(base + 5 * K, 64)],
                        s_hbm.at[cid, pl.ds(base + 5 * K, 64)])


def _ln_relu(u):
    mu = jnp.mean(u, axis=-1, keepdims=True)
    var = jnp.var(u, axis=-1, keepdims=True)
    return jax.nn.relu((u - mu) / jnp.sqrt(var + LN_EPS))


def _stage_a_body(x_ref, w_ref, dp0_ref, dp1_ref, hp_ref, hb_ref, dis_ref):
    dis = jax.lax.rsqrt(dp0_ref[...] + dp1_ref[...] + 1.0)
    g = jnp.dot(x_ref[...], w_ref[...], preferred_element_type=_f32)
    hp = g * dis
    hp_ref[...] = hp
    hb_ref[...] = hp.astype(_bf16)
    dis_ref[...] = dis


def _stage_b_body(s0_ref, s1_ref, hp_ref, dis_ref, b_ref, w2_ref,
                  hp2_ref, hb2_ref):
    s = jnp.concatenate([s0_ref[...], s1_ref[...]], axis=-1)
    u = dis_ref[...] * (s + hp_ref[...]) + b_ref[...]
    h = _ln_relu(u)
    g = jnp.dot(h, w2_ref[...], preferred_element_type=_f32)
    hp2 = g * dis_ref[...]
    hp2_ref[...] = hp2
    hb2_ref[...] = hp2.astype(_bf16)


def _stage_c_body(s0_ref, s1_ref, hp_ref, dis_ref, b_ref, o_ref):
    s = jnp.concatenate([s0_ref[...], s1_ref[...]], axis=-1)
    u = dis_ref[...] * (s + hp_ref[...]) + b_ref[...]
    o_ref[...] = _ln_relu(u)


BLK = 1000
GRID = N // BLK
_row_spec = pl.BlockSpec((BLK, D), lambda i: (i, 0))
_half_spec = pl.BlockSpec((BLK, DH), lambda i: (i, 0))
_col1_spec = pl.BlockSpec((BLK, 1), lambda i: (i, 0))
_w_spec = pl.BlockSpec((D, D), lambda i: (0, 0))
_b_spec = pl.BlockSpec((1, D), lambda i: (0, 0))


def _stage_a(x, W, dp0, dp1):
    return pl.pallas_call(
        _stage_a_body,
        grid=(GRID,),
        in_specs=[_row_spec, _w_spec, _col1_spec, _col1_spec],
        out_specs=[_row_spec, _row_spec, _col1_spec],
        out_shape=[
            jax.ShapeDtypeStruct((N, D), _f32),
            jax.ShapeDtypeStruct((N, D), _bf16),
            jax.ShapeDtypeStruct((N, 1), _f32),
        ],
    )(x, W, dp0, dp1)


def _stage_b(S0, S1, hp, dis, b, W2):
    return pl.pallas_call(
        _stage_b_body,
        grid=(GRID,),
        in_specs=[_half_spec, _half_spec, _row_spec, _col1_spec, _b_spec,
                  _w_spec],
        out_specs=[_row_spec, _row_spec],
        out_shape=[
            jax.ShapeDtypeStruct((N, D), _f32),
            jax.ShapeDtypeStruct((N, D), _bf16),
        ],
    )(S0, S1, hp, dis, b, W2)


def _stage_c(S0, S1, hp, dis, b):
    return pl.pallas_call(
        _stage_c_body,
        grid=(GRID,),
        in_specs=[_half_spec, _half_spec, _row_spec, _col1_spec, _b_spec],
        out_specs=_row_spec,
        out_shape=jax.ShapeDtypeStruct((N, D), _f32),
    )(S0, S1, hp, dis, b)


def _to_sc(hb):
    t = hb.reshape(N, 4, 2, 16).transpose(0, 1, 3, 2).reshape(N, 64, 2)
    return lax.bitcast_convert_type(t, _u32).reshape(2 * N, WH)


def kernel(x, edge_index, edge_weight, W1, b1, W2, b2):
    row = edge_index[0].astype(_i32)
    col = edge_index[1].astype(_i32)
    w = edge_weight.astype(_f32)

    pad = E_PAD - E
    rp = jnp.concatenate([row, jnp.zeros((pad,), _i32)]).reshape(16, CHUNKS, K)
    col_r = jnp.concatenate([col, jnp.zeros((pad,), _i32)]).reshape(16, CHUNKS, K)
    w_r = jnp.concatenate([w, jnp.zeros((pad,), _f32)]).reshape(16, CHUNKS, K)
    row2_r = jnp.concatenate([2 * rp, 2 * rp + 1]).reshape(32, CHUNKS, K)

    dp = _deg_sc(col_r, w_r)
    dp0 = dp[:N].reshape(N, 1)
    dp1 = dp[N:].reshape(N, 1)

    hp, hb, dis = _stage_a(x, W1, dp0, dp1)
    S = _msg_sc(_to_sc(hb), row2_r, col_r, w_r)
    hp2, hb2 = _stage_b(S[0], S[1], hp, dis, b1.reshape(1, D), W2)
    S2 = _msg_sc(_to_sc(hb2), row2_r, col_r, w_r)
    return _stage_c(S2[0], S2[1], hp2, dis, b2.reshape(1, D))


# speedup vs baseline: 1.0125x; 1.0125x over previous
import functools

import jax
import jax.numpy as jnp
from jax import lax
from jax.experimental import pallas as pl
from jax.experimental.pallas import tpu as pltpu
from jax.experimental.pallas import tpu_sc as plsc

N = 10000
D = 128
DH = D // 2
WH = DH // 2
E = 320000
LN_EPS = 1e-5

K = 112
CHUNKS = 179
EPW = CHUNKS * K
E_PAD = 16 * EPW

_f32 = jnp.float32
_i32 = jnp.int32
_u32 = jnp.uint32
_bf16 = jnp.bfloat16

_sc_mesh = plsc.VectorSubcoreMesh(core_axis_name="c", subcore_axis_name="s")


@functools.partial(
    pl.kernel,
    out_type=jax.ShapeDtypeStruct((2 * N,), _f32),
    mesh=_sc_mesh,
    scratch_types=[
        pltpu.VMEM((CHUNKS, K), _i32),
        pltpu.VMEM((CHUNKS, K), _f32),
        pltpu.VMEM((N,), _f32),
        pltpu.VMEM_SHARED((N,), _f32),
    ],
)
def _deg_sc(col_hbm, w_hbm, dp_hbm, col_v, w_v, dvmem, dacc):
    cid = lax.axis_index("c")
    sid = lax.axis_index("s")

    pltpu.sync_copy(col_hbm.at[sid], col_v)
    pltpu.sync_copy(w_hbm.at[sid], w_v)

    @pl.when(sid == 0)
    def _zero():
        zero16 = jnp.zeros((16,), _f32)

        def zr(i, _):
            dvmem[pl.ds(i * 16, 16)] = zero16
            return 0

        lax.fori_loop(0, N // 16, zr, 0)
        pltpu.sync_copy(dvmem, dacc)

    plsc.subcore_barrier()

    lo = jnp.where(cid == 0, 0, 90)
    hi = jnp.where(cid == 0, 90, CHUNKS)

    def chunk(g, _):
        pltpu.sync_copy(w_v.at[g], dacc.at[col_v.at[g]], add=True)
        return 0

    lax.fori_loop(lo, hi, chunk, 0)

    plsc.subcore_barrier()

    @pl.when(sid == 0)
    def _writeout():
        pltpu.sync_copy(dacc, dvmem)
        pltpu.sync_copy(dvmem, dp_hbm.at[pl.ds(cid * N, N)])


@functools.partial(
    pl.kernel,
    out_type=jax.ShapeDtypeStruct((2, N, DH), _f32),
    mesh=_sc_mesh,
    scratch_types=[
        pltpu.VMEM((CHUNKS, K), _i32),
        pltpu.VMEM((CHUNKS, K), _i32),
        pltpu.VMEM((CHUNKS, K), _f32),
        pltpu.VMEM((K, WH), _i32),
        pltpu.VMEM((K, WH), _i32),
        pltpu.VMEM((K, DH), _f32),
        pltpu.VMEM((K, DH), _f32),
        pltpu.VMEM_SHARED((N, DH), _f32),
        pltpu.SemaphoreType.DMA,
        pltpu.SemaphoreType.DMA,
        pltpu.SemaphoreType.DMA,
        pltpu.SemaphoreType.DMA,
    ],
    compiler_params=pltpu.CompilerParams(use_tc_tiling_on_sc=False),
)
def _msg_sc(hp_hbm, row_hbm, col_hbm, w_hbm, s_hbm,
            row_v, col_v, w_v, g0, g1, s0, s1, acc,
            sem_g0, sem_g1, sem_s0, sem_s1):
    cid = lax.axis_index("c")
    sid = lax.axis_index("s")
    wid = cid * 16 + sid

    pltpu.sync_copy(row_hbm.at[wid], row_v)
    pltpu.sync_copy(col_hbm.at[sid], col_v)
    pltpu.sync_copy(w_hbm.at[sid], w_v)

    zero16 = jnp.zeros((16,), _f32)

    def zr(i, _):
        for q in range(DH // 16):
            s0[i, pl.ds(q * 16, 16)] = zero16
        return 0

    lax.fori_loop(0, K, zr, 0)
    base = sid * 624
    for j in range(5):
        pltpu.sync_copy(s0, acc.at[pl.ds(base + j * K, K)])

    @pl.when(sid == 15)
    def _ztail_full():
        pltpu.sync_copy(s0.at[pl.ds(0, 80)], acc.at[pl.ds(base + 5 * K, 80)])

    @pl.when(sid != 15)
    def _ztail_part():
        pltpu.sync_copy(s0.at[pl.ds(0, 64)], acc.at[pl.ds(base + 5 * K, 64)])

    plsc.subcore_barrier()

    mask_hi = jnp.full((16,), -65536, dtype=_i32)

    def scale(gb, sb, g):
        def grp(j, _):
            w16 = w_v[g, pl.ds(j * 16, 16)]
            for t in range(16):
                e = j * 16 + t
                sw = w16[t]
                for q in range(WH // 16):
                    v = gb[e, pl.ds(q * 16, 16)]
                    fa = lax.bitcast_convert_type(v << 16, _f32)
                    fb = lax.bitcast_convert_type(v & mask_hi, _f32)
                    sb[e, pl.ds(q * 32, 16)] = fa * sw
                    sb[e, pl.ds(q * 32 + 16, 16)] = fb * sw
            return 0

        lax.fori_loop(0, K // 16, grp, 0)

    pltpu.async_copy(hp_hbm.at[row_v.at[0]], g0, sem_g0)
    pltpu.async_copy(hp_hbm.at[row_v.at[1]], g1, sem_g1)

    def pipe(i, _):
        ga = 2 * i
        gb_ = 2 * i + 1

        pltpu.make_async_copy(hp_hbm.at[row_v.at[ga]], g0, sem_g0).wait()

        @pl.when(i >= 1)
        def _drain_a():
            pltpu.make_async_copy(s0, acc.at[col_v.at[ga - 2]], sem_s0).wait()

        scale(g0, s0, ga)
        pltpu.async_copy(s0, acc.at[col_v.at[ga]], sem_s0, add=True)
        pltpu.async_copy(hp_hbm.at[row_v.at[ga + 2]], g0, sem_g0)

        pltpu.make_async_copy(hp_hbm.at[row_v.at[gb_]], g1, sem_g1).wait()

        @pl.when(i >= 1)
        def _drain_b():
            pltpu.make_async_copy(s1, acc.at[col_v.at[gb_ - 2]], sem_s1).wait()

        scale(g1, s1, gb_)
        pltpu.async_copy(s1, acc.at[col_v.at[gb_]], sem_s1, add=True)

        @pl.when(i < (CHUNKS - 1) // 2 - 1)
        def _next_b():
            pltpu.async_copy(hp_hbm.at[row_v.at[gb_ + 2]], g1, sem_g1)

        return 0

    lax.fori_loop(0, (CHUNKS - 1) // 2, pipe, 0)

    last = CHUNKS - 1
    pltpu.make_async_copy(hp_hbm.at[row_v.at[last]], g0, sem_g0).wait()
    pltpu.make_async_copy(s0, acc.at[col_v.at[last - 2]], sem_s0).wait()
    scale(g0, s0, last)
    pltpu.async_copy(s0, acc.at[col_v.at[last]], sem_s0, add=True)
    pltpu.make_async_copy(s1, acc.at[col_v.at[last - 1]], sem_s1).wait()
    pltpu.make_async_copy(s0, acc.at[col_v.at[last]], sem_s0).wait()

    plsc.subcore_barrier()

    for j in range(5):
        pltpu.sync_copy(acc.at[pl.ds(base + j * K, K)],
                        s_hbm.at[cid, pl.ds(base + j * K, K)])

    @pl.when(sid == 15)
    def _wtail_full():
        pltpu.sync_copy(acc.at[pl.ds(base + 5 * K, 80)],
                        s_hbm.at[cid, pl.ds(base + 5 * K, 80)])

    @pl.when(sid != 15)
    def _wtail_part():
        pltpu.sync_copy(acc.at[pl.ds(base + 5 * K, 64)],
                        s_hbm.at[cid, pl.ds(base + 5 * K, 64)])


def _ln_relu(u):
    mu = jnp.mean(u, axis=-1, keepdims=True)
    var = jnp.var(u, axis=-1, keepdims=True)
    return jax.nn.relu((u - mu) / jnp.sqrt(var + LN_EPS))


def _stage_a_body(x_ref, w_ref, dp0_ref, dp1_ref, hp_ref, hb_ref, dis_ref):
    dis = jax.lax.rsqrt(dp0_ref[...] + dp1_ref[...] + 1.0)
    g = jnp.dot(x_ref[...], w_ref[...], preferred_element_type=_f32)
    hp = g * dis
    hp_ref[...] = hp
    hb_ref[...] = hp.astype(_bf16)
    dis_ref[...] = dis


def _stage_b_body(s0_ref, s1_ref, hp_ref, dis_ref, b_ref, w2_ref,
                  hp2_ref, hb2_ref):
    s = jnp.concatenate([s0_ref[...], s1_ref[...]], axis=-1)
    u = dis_ref[...] * (s + hp_ref[...]) + b_ref[...]
    h = _ln_relu(u)
    g = jnp.dot(h, w2_ref[...], preferred_element_type=_f32)
    hp2 = g * dis_ref[...]
    hp2_ref[...] = hp2
    hb2_ref[...] = hp2.astype(_bf16)


def _stage_c_body(s0_ref, s1_ref, hp_ref, dis_ref, b_ref, o_ref):
    s = jnp.concatenate([s0_ref[...], s1_ref[...]], axis=-1)
    u = dis_ref[...] * (s + hp_ref[...]) + b_ref[...]
    o_ref[...] = _ln_relu(u)


BLK = 1000
GRID = N // BLK
_row_spec = pl.BlockSpec((BLK, D), lambda i: (i, 0))
_half_spec = pl.BlockSpec((BLK, DH), lambda i: (i, 0))
_col1_spec = pl.BlockSpec((BLK, 1), lambda i: (i, 0))
_w_spec = pl.BlockSpec((D, D), lambda i: (0, 0))
_b_spec = pl.BlockSpec((1, D), lambda i: (0, 0))


def _stage_a(x, W, dp0, dp1):
    return pl.pallas_call(
        _stage_a_body,
        grid=(GRID,),
        in_specs=[_row_spec, _w_spec, _col1_spec, _col1_spec],
        out_specs=[_row_spec, _row_spec, _col1_spec],
        out_shape=[
            jax.ShapeDtypeStruct((N, D), _f32),
            jax.ShapeDtypeStruct((N, D), _bf16),
            jax.ShapeDtypeStruct((N, 1), _f32),
        ],
    )(x, W, dp0, dp1)


def _stage_b(S0, S1, hp, dis, b, W2):
    return pl.pallas_call(
        _stage_b_body,
        grid=(GRID,),
        in_specs=[_half_spec, _half_spec, _row_spec, _col1_spec, _b_spec,
                  _w_spec],
        out_specs=[_row_spec, _row_spec],
        out_shape=[
            jax.ShapeDtypeStruct((N, D), _f32),
            jax.ShapeDtypeStruct((N, D), _bf16),
        ],
    )(S0, S1, hp, dis, b, W2)


def _stage_c(S0, S1, hp, dis, b):
    return pl.pallas_call(
        _stage_c_body,
        grid=(GRID,),
        in_specs=[_half_spec, _half_spec, _row_spec, _col1_spec, _b_spec],
        out_specs=_row_spec,
        out_shape=jax.ShapeDtypeStruct((N, D), _f32),
    )(S0, S1, hp, dis, b)


def _to_sc(hb):
    t = hb.reshape(N, 4, 2, 16).transpose(0, 1, 3, 2).reshape(N, 64, 2)
    return lax.bitcast_convert_type(t, _i32).reshape(2 * N, WH)


def kernel(x, edge_index, edge_weight, W1, b1, W2, b2):
    row = edge_index[0].astype(_i32)
    col = edge_index[1].astype(_i32)
    w = edge_weight.astype(_f32)

    pad = E_PAD - E
    rp = jnp.concatenate([row, jnp.zeros((pad,), _i32)]).reshape(16, CHUNKS, K)
    col_r = jnp.concatenate([col, jnp.zeros((pad,), _i32)]).reshape(16, CHUNKS, K)
    w_r = jnp.concatenate([w, jnp.zeros((pad,), _f32)]).reshape(16, CHUNKS, K)
    row2_r = jnp.concatenate([2 * rp, 2 * rp + 1]).reshape(32, CHUNKS, K)

    dp = _deg_sc(col_r, w_r)
    dp0 = dp[:N].reshape(N, 1)
    dp1 = dp[N:].reshape(N, 1)

    hp, hb, dis = _stage_a(x, W1, dp0, dp1)
    S = _msg_sc(_to_sc(hb), row2_r, col_r, w_r)
    hp2, hb2 = _stage_b(S[0], S[1], hp, dis, b1.reshape(1, D), W2)
    S2 = _msg_sc(_to_sc(hb2), row2_r, col_r, w_r)
    return _stage_c(S2[0], S2[1], hp2, dis, b2.reshape(1, D))


# speedup vs baseline: 1.6273x; 1.6073x over previous
import functools

import jax
import jax.numpy as jnp
from jax import lax
from jax.experimental import pallas as pl
from jax.experimental.pallas import tpu as pltpu
from jax.experimental.pallas import tpu_sc as plsc

N = 10000
D = 128
DH = D // 2
E = 320000
LN_EPS = 1e-5

K = 112
CHUNKS = 179
EPW = CHUNKS * K
E_PAD = 16 * EPW

_f32 = jnp.float32
_i32 = jnp.int32

_sc_mesh = plsc.VectorSubcoreMesh(core_axis_name="c", subcore_axis_name="s")


@functools.partial(
    pl.kernel,
    out_type=jax.ShapeDtypeStruct((2 * N,), _f32),
    mesh=_sc_mesh,
    scratch_types=[
        pltpu.VMEM((CHUNKS, K), _i32),
        pltpu.VMEM((CHUNKS, K), _f32),
        pltpu.VMEM((N,), _f32),
        pltpu.VMEM_SHARED((N,), _f32),
    ],
)
def _deg_sc(col_hbm, w_hbm, dp_hbm, col_v, w_v, dvmem, dacc):
    cid = lax.axis_index("c")
    sid = lax.axis_index("s")

    pltpu.sync_copy(col_hbm.at[sid], col_v)
    pltpu.sync_copy(w_hbm.at[sid], w_v)

    @pl.when(sid == 0)
    def _zero():
        zero16 = jnp.zeros((16,), _f32)

        def zr(i, _):
            dvmem[pl.ds(i * 16, 16)] = zero16
            return 0

        lax.fori_loop(0, N // 16, zr, 0)
        pltpu.sync_copy(dvmem, dacc)

    plsc.subcore_barrier()

    lo = jnp.where(cid == 0, 0, 90)
    hi = jnp.where(cid == 0, 90, CHUNKS)

    def chunk(g, _):
        pltpu.sync_copy(w_v.at[g], dacc.at[col_v.at[g]], add=True)
        return 0

    lax.fori_loop(lo, hi, chunk, 0)

    plsc.subcore_barrier()

    @pl.when(sid == 0)
    def _writeout():
        pltpu.sync_copy(dacc, dvmem)
        pltpu.sync_copy(dvmem, dp_hbm.at[pl.ds(cid * N, N)])


@functools.partial(
    pl.kernel,
    out_type=jax.ShapeDtypeStruct((2, N, DH), _f32),
    mesh=_sc_mesh,
    scratch_types=[
        pltpu.VMEM((CHUNKS, K), _i32),
        pltpu.VMEM((CHUNKS, K), _i32),
        pltpu.VMEM((CHUNKS, K), _f32),
        pltpu.VMEM((K, DH), _f32),
        pltpu.VMEM((K, DH), _f32),
        pltpu.VMEM((K, DH), _f32),
        pltpu.VMEM((K, DH), _f32),
        pltpu.VMEM_SHARED((N, DH), _f32),
        pltpu.SemaphoreType.DMA,
        pltpu.SemaphoreType.DMA,
        pltpu.SemaphoreType.DMA,
        pltpu.SemaphoreType.DMA,
    ],
    compiler_params=pltpu.CompilerParams(use_tc_tiling_on_sc=False),
)
def _msg_sc(hp_hbm, row_hbm, col_hbm, w_hbm, s_hbm,
            row_v, col_v, w_v, g0, g1, s0, s1, acc,
            sem_g0, sem_g1, sem_s0, sem_s1):
    cid = lax.axis_index("c")
    sid = lax.axis_index("s")
    wid = cid * 16 + sid

    pltpu.sync_copy(row_hbm.at[wid], row_v)
    pltpu.sync_copy(col_hbm.at[sid], col_v)
    pltpu.sync_copy(w_hbm.at[sid], w_v)

    zero16 = jnp.zeros((16,), _f32)

    def zr(i, _):
        for q in range(DH // 16):
            s0[i, pl.ds(q * 16, 16)] = zero16
        return 0

    lax.fori_loop(0, K, zr, 0)
    base = sid * 624
    for j in range(5):
        pltpu.sync_copy(s0, acc.at[pl.ds(base + j * K, K)])

    @pl.when(sid == 15)
    def _ztail_full():
        pltpu.sync_copy(s0.at[pl.ds(0, 80)], acc.at[pl.ds(base + 5 * K, 80)])

    @pl.when(sid != 15)
    def _ztail_part():
        pltpu.sync_copy(s0.at[pl.ds(0, 64)], acc.at[pl.ds(base + 5 * K, 64)])

    plsc.subcore_barrier()

    def scale(gb, sb, g):
        def grp(j, _):
            w16 = w_v[g, pl.ds(j * 16, 16)]
            for t in range(16):
                e = j * 16 + t
                sw = w16[t]
                for q in range(DH // 16):
                    sl = pl.ds(q * 16, 16)
                    sb[e, sl] = gb[e, sl] * sw
            return 0

        lax.fori_loop(0, K // 16, grp, 0)

    pltpu.async_copy(hp_hbm.at[row_v.at[0]], g0, sem_g0)
    pltpu.async_copy(hp_hbm.at[row_v.at[1]], g1, sem_g1)

    def pipe(i, _):
        ga = 2 * i
        gb_ = 2 * i + 1

        pltpu.make_async_copy(hp_hbm.at[row_v.at[ga]], g0, sem_g0).wait()

        @pl.when(i >= 1)
        def _drain_a():
            pltpu.make_async_copy(s0, acc.at[col_v.at[ga - 2]], sem_s0).wait()

        scale(g0, s0, ga)
        pltpu.async_copy(s0, acc.at[col_v.at[ga]], sem_s0, add=True)
        pltpu.async_copy(hp_hbm.at[row_v.at[ga + 2]], g0, sem_g0)

        pltpu.make_async_copy(hp_hbm.at[row_v.at[gb_]], g1, sem_g1).wait()

        @pl.when(i >= 1)
        def _drain_b():
            pltpu.make_async_copy(s1, acc.at[col_v.at[gb_ - 2]], sem_s1).wait()

        scale(g1, s1, gb_)
        pltpu.async_copy(s1, acc.at[col_v.at[gb_]], sem_s1, add=True)

        @pl.when(i < (CHUNKS - 1) // 2 - 1)
        def _next_b():
            pltpu.async_copy(hp_hbm.at[row_v.at[gb_ + 2]], g1, sem_g1)

        return 0

    lax.fori_loop(0, (CHUNKS - 1) // 2, pipe, 0)

    last = CHUNKS - 1
    pltpu.make_async_copy(hp_hbm.at[row_v.at[last]], g0, sem_g0).wait()
    pltpu.make_async_copy(s0, acc.at[col_v.at[last - 2]], sem_s0).wait()
    scale(g0, s0, last)
    pltpu.async_copy(s0, acc.at[col_v.at[last]], sem_s0, add=True)
    pltpu.make_async_copy(s1, acc.at[col_v.at[last - 1]], sem_s1).wait()
    pltpu.make_async_copy(s0, acc.at[col_v.at[last]], sem_s0).wait()

    plsc.subcore_barrier()

    for j in range(5):
        pltpu.sync_copy(acc.at[pl.ds(base + j * K, K)],
                        s_hbm.at[cid, pl.ds(base + j * K, K)])

    @pl.when(sid == 15)
    def _wtail_full():
        pltpu.sync_copy(acc.at[pl.ds(base + 5 * K, 80)],
                        s_hbm.at[cid, pl.ds(base + 5 * K, 80)])

    @pl.when(sid != 15)
    def _wtail_part():
        pltpu.sync_copy(acc.at[pl.ds(base + 5 * K, 64)],
                        s_hbm.at[cid, pl.ds(base + 5 * K, 64)])


def _ln_relu(u):
    mu = jnp.mean(u, axis=-1, keepdims=True)
    var = jnp.var(u, axis=-1, keepdims=True)
    return jax.nn.relu((u - mu) / jnp.sqrt(var + LN_EPS))


def _stage_a_body(x_ref, w_ref, dp0_ref, dp1_ref, hp_ref, dis_ref):
    dis = jax.lax.rsqrt(dp0_ref[...] + dp1_ref[...] + 1.0)
    g = jnp.dot(x_ref[...], w_ref[...], preferred_element_type=_f32)
    hp_ref[...] = g * dis
    dis_ref[...] = dis


def _stage_b_body(s0_ref, s1_ref, hp_ref, dis_ref, b_ref, w2_ref, hp2_ref):
    s = jnp.concatenate([s0_ref[...], s1_ref[...]], axis=-1)
    u = dis_ref[...] * (s + hp_ref[...]) + b_ref[...]
    h = _ln_relu(u)
    g = jnp.dot(h, w2_ref[...], preferred_element_type=_f32)
    hp2_ref[...] = g * dis_ref[...]


def _stage_c_body(s0_ref, s1_ref, hp_ref, dis_ref, b_ref, o_ref):
    s = jnp.concatenate([s0_ref[...], s1_ref[...]], axis=-1)
    u = dis_ref[...] * (s + hp_ref[...]) + b_ref[...]
    o_ref[...] = _ln_relu(u)


BLK = 1000
GRID = N // BLK
_row_spec = pl.BlockSpec((BLK, D), lambda i: (i, 0))
_half_spec = pl.BlockSpec((BLK, DH), lambda i: (i, 0))
_col1_spec = pl.BlockSpec((BLK, 1), lambda i: (i, 0))
_w_spec = pl.BlockSpec((D, D), lambda i: (0, 0))
_b_spec = pl.BlockSpec((1, D), lambda i: (0, 0))


def _stage_a(x, W, dp0, dp1):
    return pl.pallas_call(
        _stage_a_body,
        grid=(GRID,),
        in_specs=[_row_spec, _w_spec, _col1_spec, _col1_spec],
        out_specs=[_row_spec, _col1_spec],
        out_shape=[
            jax.ShapeDtypeStruct((N, D), _f32),
            jax.ShapeDtypeStruct((N, 1), _f32),
        ],
    )(x, W, dp0, dp1)


def _stage_b(S0, S1, hp, dis, b, W2):
    return pl.pallas_call(
        _stage_b_body,
        grid=(GRID,),
        in_specs=[_half_spec, _half_spec, _row_spec, _col1_spec, _b_spec,
                  _w_spec],
        out_specs=_row_spec,
        out_shape=jax.ShapeDtypeStruct((N, D), _f32),
    )(S0, S1, hp, dis, b, W2)


def _stage_c(S0, S1, hp, dis, b):
    return pl.pallas_call(
        _stage_c_body,
        grid=(GRID,),
        in_specs=[_half_spec, _half_spec, _row_spec, _col1_spec, _b_spec],
        out_specs=_row_spec,
        out_shape=jax.ShapeDtypeStruct((N, D), _f32),
    )(S0, S1, hp, dis, b)


def kernel(x, edge_index, edge_weight, W1, b1, W2, b2):
    row = edge_index[0].astype(_i32)
    col = edge_index[1].astype(_i32)
    w = edge_weight.astype(_f32)

    pad = E_PAD - E
    rp = jnp.concatenate([row, jnp.zeros((pad,), _i32)]).reshape(16, CHUNKS, K)
    col_r = jnp.concatenate([col, jnp.zeros((pad,), _i32)]).reshape(16, CHUNKS, K)
    w_r = jnp.concatenate([w, jnp.zeros((pad,), _f32)]).reshape(16, CHUNKS, K)
    row2_r = jnp.concatenate([2 * rp, 2 * rp + 1]).reshape(32, CHUNKS, K)

    dp = _deg_sc(col_r, w_r)
    dp0 = dp[:N].reshape(N, 1)
    dp1 = dp[N:].reshape(N, 1)

    hp, dis = _stage_a(x, W1, dp0, dp1)
    S = _msg_sc(hp.reshape(2 * N, DH), row2_r, col_r, w_r)
    hp2 = _stage_b(S[0], S[1], hp, dis, b1.reshape(1, D), W2)
    S2 = _msg_sc(hp2.reshape(2 * N, DH), row2_r, col_r, w_r)
    return _stage_c(S2[0], S2[1], hp2, dis, b2.reshape(1, D))


# speedup vs baseline: 1.6313x; 1.0024x over previous
import functools

import jax
import jax.numpy as jnp
from jax import lax
from jax.experimental import pallas as pl
from jax.experimental.pallas import tpu as pltpu
from jax.experimental.pallas import tpu_sc as plsc

N = 10000
D = 128
DH = D // 2
E = 320000
LN_EPS = 1e-5

K = 112
CHUNKS = 179
EPW = CHUNKS * K
E_PAD = 16 * EPW

_f32 = jnp.float32
_i32 = jnp.int32

_sc_mesh = plsc.VectorSubcoreMesh(core_axis_name="c", subcore_axis_name="s")


@functools.partial(
    pl.kernel,
    out_type=jax.ShapeDtypeStruct((2 * N,), _f32),
    mesh=_sc_mesh,
    scratch_types=[
        pltpu.VMEM((CHUNKS, K), _i32),
        pltpu.VMEM((CHUNKS, K), _f32),
        pltpu.VMEM((N,), _f32),
        pltpu.VMEM_SHARED((N,), _f32),
    ],
)
def _deg_sc(col_hbm, w_hbm, dp_hbm, col_v, w_v, dvmem, dacc):
    cid = lax.axis_index("c")
    sid = lax.axis_index("s")

    pltpu.sync_copy(col_hbm.at[sid], col_v)
    pltpu.sync_copy(w_hbm.at[sid], w_v)

    @pl.when(sid == 0)
    def _zero():
        zero16 = jnp.zeros((16,), _f32)

        def zr(i, _):
            dvmem[pl.ds(i * 16, 16)] = zero16
            return 0

        lax.fori_loop(0, N // 16, zr, 0)
        pltpu.sync_copy(dvmem, dacc)

    plsc.subcore_barrier()

    lo = jnp.where(cid == 0, 0, 90)
    hi = jnp.where(cid == 0, 90, CHUNKS)

    def chunk(g, _):
        pltpu.sync_copy(w_v.at[g], dacc.at[col_v.at[g]], add=True)
        return 0

    lax.fori_loop(lo, hi, chunk, 0)

    plsc.subcore_barrier()

    @pl.when(sid == 0)
    def _writeout():
        pltpu.sync_copy(dacc, dvmem)
        pltpu.sync_copy(dvmem, dp_hbm.at[pl.ds(cid * N, N)])


@functools.partial(
    pl.kernel,
    out_type=jax.ShapeDtypeStruct((2, N, DH), _f32),
    mesh=_sc_mesh,
    scratch_types=[
        pltpu.VMEM((CHUNKS, K), _i32),
        pltpu.VMEM((CHUNKS, K), _i32),
        pltpu.VMEM((CHUNKS, K), _f32),
        pltpu.VMEM((K, DH), _f32),
        pltpu.VMEM((K, DH), _f32),
        pltpu.VMEM((K, DH), _f32),
        pltpu.VMEM((K, DH), _f32),
        pltpu.VMEM_SHARED((N, DH), _f32),
        pltpu.SemaphoreType.DMA,
        pltpu.SemaphoreType.DMA,
        pltpu.SemaphoreType.DMA,
        pltpu.SemaphoreType.DMA,
    ],
    compiler_params=pltpu.CompilerParams(use_tc_tiling_on_sc=False),
)
def _msg_sc(hp_hbm, row_hbm, col_hbm, w_hbm, s_hbm,
            row_v, col_v, w_v, g0, g1, s0, s1, acc,
            sem_g0, sem_g1, sem_s0, sem_s1):
    cid = lax.axis_index("c")
    sid = lax.axis_index("s")
    wid = cid * 16 + sid

    pltpu.sync_copy(row_hbm.at[wid], row_v)
    pltpu.sync_copy(col_hbm.at[sid], col_v)
    pltpu.sync_copy(w_hbm.at[sid], w_v)

    zero16 = jnp.zeros((16,), _f32)

    def zr(i, _):
        for q in range(DH // 16):
            s0[i, pl.ds(q * 16, 16)] = zero16
        return 0

    lax.fori_loop(0, K, zr, 0)
    base = sid * 624
    for j in range(5):
        pltpu.sync_copy(s0, acc.at[pl.ds(base + j * K, K)])

    @pl.when(sid == 15)
    def _ztail_full():
        pltpu.sync_copy(s0.at[pl.ds(0, 80)], acc.at[pl.ds(base + 5 * K, 80)])

    @pl.when(sid != 15)
    def _ztail_part():
        pltpu.sync_copy(s0.at[pl.ds(0, 64)], acc.at[pl.ds(base + 5 * K, 64)])

    plsc.subcore_barrier()

    def scale(gb, sb, g):
        def grp(j, _):
            w16 = w_v[g, pl.ds(j * 16, 16)]
            for t in range(16):
                e = j * 16 + t
                sw = w16[t]
                for q in range(DH // 16):
                    sl = pl.ds(q * 16, 16)
                    sb[e, sl] = gb[e, sl] * sw
            return 0

        lax.fori_loop(0, K // 16, grp, 0)

    pltpu.async_copy(hp_hbm.at[row_v.at[0]], g0, sem_g0)
    pltpu.async_copy(hp_hbm.at[row_v.at[1]], g1, sem_g1)

    def pipe(i, _):
        ga = 2 * i
        gb_ = 2 * i + 1

        pltpu.make_async_copy(hp_hbm.at[row_v.at[ga]], g0, sem_g0).wait()

        @pl.when(i >= 1)
        def _drain_a():
            pltpu.make_async_copy(s0, acc.at[col_v.at[ga - 2]], sem_s0).wait()

        scale(g0, s0, ga)
        pltpu.async_copy(s0, acc.at[col_v.at[ga]], sem_s0, add=True)
        pltpu.async_copy(hp_hbm.at[row_v.at[ga + 2]], g0, sem_g0)

        pltpu.make_async_copy(hp_hbm.at[row_v.at[gb_]], g1, sem_g1).wait()

        @pl.when(i >= 1)
        def _drain_b():
            pltpu.make_async_copy(s1, acc.at[col_v.at[gb_ - 2]], sem_s1).wait()

        scale(g1, s1, gb_)
        pltpu.async_copy(s1, acc.at[col_v.at[gb_]], sem_s1, add=True)

        @pl.when(i < (CHUNKS - 1) // 2 - 1)
        def _next_b():
            pltpu.async_copy(hp_hbm.at[row_v.at[gb_ + 2]], g1, sem_g1)

        return 0

    lax.fori_loop(0, (CHUNKS - 1) // 2, pipe, 0)

    last = CHUNKS - 1
    pltpu.make_async_copy(hp_hbm.at[row_v.at[last]], g0, sem_g0).wait()
    pltpu.make_async_copy(s0, acc.at[col_v.at[last - 2]], sem_s0).wait()
    scale(g0, s0, last)
    pltpu.async_copy(s0, acc.at[col_v.at[last]], sem_s0, add=True)
    pltpu.make_async_copy(s1, acc.at[col_v.at[last - 1]], sem_s1).wait()
    pltpu.make_async_copy(s0, acc.at[col_v.at[last]], sem_s0).wait()

    plsc.subcore_barrier()

    for j in range(5):
        pltpu.sync_copy(acc.at[pl.ds(base + j * K, K)],
                        s_hbm.at[cid, pl.ds(base + j * K, K)])

    @pl.when(sid == 15)
    def _wtail_full():
        pltpu.sync_copy(acc.at[pl.ds(base + 5 * K, 80)],
                        s_hbm.at[cid, pl.ds(base + 5 * K, 80)])

    @pl.when(sid != 15)
    def _wtail_part():
        pltpu.sync_copy(acc.at[pl.ds(base + 5 * K, 64)],
                        s_hbm.at[cid, pl.ds(base + 5 * K, 64)])


def _ln_relu(u):
    mu = jnp.mean(u, axis=-1, keepdims=True)
    var = jnp.var(u, axis=-1, keepdims=True)
    return jax.nn.relu((u - mu) / jnp.sqrt(var + LN_EPS))


def _matmul_body(x_ref, w_ref, g_ref):
    g_ref[...] = jnp.dot(x_ref[...], w_ref[...], preferred_element_type=_f32)


def _stage_a_body(g_ref, dp0_ref, dp1_ref, hp_ref, dis_ref):
    dis = jax.lax.rsqrt(dp0_ref[...] + dp1_ref[...] + 1.0)
    hp_ref[...] = g_ref[...] * dis
    dis_ref[...] = dis


def _stage_b_body(s0_ref, s1_ref, hp_ref, dis_ref, b_ref, w2_ref, hp2_ref):
    s = jnp.concatenate([s0_ref[...], s1_ref[...]], axis=-1)
    u = dis_ref[...] * (s + hp_ref[...]) + b_ref[...]
    h = _ln_relu(u)
    g = jnp.dot(h, w2_ref[...], preferred_element_type=_f32)
    hp2_ref[...] = g * dis_ref[...]


def _stage_c_body(s0_ref, s1_ref, hp_ref, dis_ref, b_ref, o_ref):
    s = jnp.concatenate([s0_ref[...], s1_ref[...]], axis=-1)
    u = dis_ref[...] * (s + hp_ref[...]) + b_ref[...]
    o_ref[...] = _ln_relu(u)


BLK = 1000
GRID = N // BLK
_row_spec = pl.BlockSpec((BLK, D), lambda i: (i, 0))
_half_spec = pl.BlockSpec((BLK, DH), lambda i: (i, 0))
_col1_spec = pl.BlockSpec((BLK, 1), lambda i: (i, 0))
_w_spec = pl.BlockSpec((D, D), lambda i: (0, 0))
_b_spec = pl.BlockSpec((1, D), lambda i: (0, 0))


def _matmul(x, W):
    return pl.pallas_call(
        _matmul_body,
        grid=(GRID,),
        in_specs=[_row_spec, _w_spec],
        out_specs=_row_spec,
        out_shape=jax.ShapeDtypeStruct((N, D), _f32),
    )(x, W)


def _stage_a(g, dp0, dp1):
    return pl.pallas_call(
        _stage_a_body,
        grid=(GRID,),
        in_specs=[_row_spec, _col1_spec, _col1_spec],
        out_specs=[_row_spec, _col1_spec],
        out_shape=[
            jax.ShapeDtypeStruct((N, D), _f32),
            jax.ShapeDtypeStruct((N, 1), _f32),
        ],
    )(g, dp0, dp1)


def _stage_b(S0, S1, hp, dis, b, W2):
    return pl.pallas_call(
        _stage_b_body,
        grid=(GRID,),
        in_specs=[_half_spec, _half_spec, _row_spec, _col1_spec, _b_spec,
                  _w_spec],
        out_specs=_row_spec,
        out_shape=jax.ShapeDtypeStruct((N, D), _f32),
    )(S0, S1, hp, dis, b, W2)


def _stage_c(S0, S1, hp, dis, b):
    return pl.pallas_call(
        _stage_c_body,
        grid=(GRID,),
        in_specs=[_half_spec, _half_spec, _row_spec, _col1_spec, _b_spec],
        out_specs=_row_spec,
        out_shape=jax.ShapeDtypeStruct((N, D), _f32),
    )(S0, S1, hp, dis, b)


def kernel(x, edge_index, edge_weight, W1, b1, W2, b2):
    row = edge_index[0].astype(_i32)
    col = edge_index[1].astype(_i32)
    w = edge_weight.astype(_f32)

    pad = E_PAD - E
    rp = jnp.concatenate([row, jnp.zeros((pad,), _i32)]).reshape(16, CHUNKS, K)
    col_r = jnp.concatenate([col, jnp.zeros((pad,), _i32)]).reshape(16, CHUNKS, K)
    w_r = jnp.concatenate([w, jnp.zeros((pad,), _f32)]).reshape(16, CHUNKS, K)
    row2_r = jnp.concatenate([2 * rp, 2 * rp + 1]).reshape(32, CHUNKS, K)

    g1 = _matmul(x, W1)
    dp = _deg_sc(col_r, w_r)
    dp0 = dp[:N].reshape(N, 1)
    dp1 = dp[N:].reshape(N, 1)

    hp, dis = _stage_a(g1, dp0, dp1)
    S = _msg_sc(hp.reshape(2 * N, DH), row2_r, col_r, w_r)
    hp2 = _stage_b(S[0], S[1], hp, dis, b1.reshape(1, D), W2)
    S2 = _msg_sc(hp2.reshape(2 * N, DH), row2_r, col_r, w_r)
    return _stage_c(S2[0], S2[1], hp2, dis, b2.reshape(1, D))
